# baseline (speedup 1.0000x reference)
import functools
import jax
import jax.numpy as jnp
from jax import lax
from jax.experimental import pallas as pl
from jax.experimental.pallas import tpu as pltpu
from jax.experimental.pallas import tpu_sc as plsc

_NC = 2
_NS = 16
_C = 80


def _edge_mlp(ea, We1, be1, We2, be2):
    E, F = ea.shape
    D = We1.shape[1]
    BE = 2000
    assert E % BE == 0

    def body(ea_ref, w1_ref, b1_ref, w2_ref, b2_ref, o_ref):
        ea_b = ea_ref[...]
        u = jnp.broadcast_to(b1_ref[...], (BE, D))
        for k in range(F):
            u = u + ea_b[:, k:k + 1] * w1_ref[k:k + 1, :]
        h = jnp.maximum(u, 0.0)
        o_ref[...] = (
            jnp.dot(h, w2_ref[...], preferred_element_type=jnp.float32)
            + b2_ref[...]
        )

    return pl.pallas_call(
        body,
        grid=(E // BE,),
        in_specs=[
            pl.BlockSpec((BE, F), lambda i: (i, 0)),
            pl.BlockSpec((F, D), lambda i: (0, 0)),
            pl.BlockSpec((1, D), lambda i: (0, 0)),
            pl.BlockSpec((D, D), lambda i: (0, 0)),
            pl.BlockSpec((1, D), lambda i: (0, 0)),
        ],
        out_specs=pl.BlockSpec((BE, D), lambda i: (i, 0)),
        out_shape=jax.ShapeDtypeStruct((E, D), jnp.float32),
    )(ea, We1, be1.reshape(1, D), We2, be2.reshape(1, D))


def _npad(n):
    return ((n + 2048 - 1) // 2048) * 2048


def _sc_pass(data, iiv, jjv, NPAD):
    gather = jjv is not None
    D = data.shape[1]
    NCH, one, C = iiv.shape
    NW = _NC * _NS
    CHW = NCH // NW
    EW = CHW * C
    stripe = NPAD // _NS
    assert stripe % C == 0
    mesh = plsc.VectorSubcoreMesh(core_axis_name="c", subcore_axis_name="s")
    nidx = 2 if gather else 1

    in_arrays = (data, iiv) + ((jjv,) if gather else ())

    @functools.partial(
        pl.kernel,
        out_type=jax.ShapeDtypeStruct((_NC * NPAD, D), jnp.float32),
        mesh=mesh,
        scratch_types=[pltpu.VMEM((nidx, C), jnp.int32)] * 4
        + [pltpu.VMEM((C, D), jnp.float32)] * 2
        + [pltpu.VMEM_SHARED((NPAD, D), jnp.float32)]
        + [pltpu.SemaphoreType.DMA] * 8,
    )
    def k(*refs):
        if gather:
            data_hbm, ii_hbm, jj_hbm = refs[0], refs[1], refs[2]
            part_hbm = refs[3]
            rest = refs[4:]
        else:
            data_hbm, ii_hbm = refs[0], refs[1]
            part_hbm = refs[2]
            rest = refs[3:]
        b = list(rest[0:4])
        g = list(rest[4:6])
        acc = rest[6]
        si = list(rest[7:11])
        sd = list(rest[11:13])
        ts = list(rest[13:15])

        cid = lax.axis_index("c")
        sid = lax.axis_index("s")
        wid = sid * _NC + cid

        def zrow(r, carry):
            for kk in range(D // 16):
                g[0][r, pl.ds(kk * 16, 16)] = jnp.zeros((16,), jnp.float32)
            return carry
        lax.fori_loop(0, C, zrow, 0)
        for q in range(stripe // C):
            pltpu.sync_copy(g[0], acc.at[pl.ds(sid * stripe + q * C, C)])
        plsc.subcore_barrier()

        def stage_idx(t, ir):
            pltpu.async_copy(ii_hbm.at[wid * CHW + t], b[ir].at[pl.ds(0, 1)],
                             si[ir])
            if gather:
                pltpu.async_copy(jj_hbm.at[wid * CHW + t],
                                 b[ir].at[pl.ds(1, 1)], si[ir])

        def wait_idx(ir):
            for _ in range(nidx):
                pltpu.make_async_copy(ii_hbm.at[0], b[ir].at[pl.ds(0, 1)],
                                      si[ir]).wait()

        def stage_data(t, sl, ir):
            if gather:
                pltpu.async_copy(data_hbm.at[b[ir].at[1]], g[sl], sd[sl])
            else:
                pltpu.async_copy(data_hbm.at[pl.ds(wid * EW + t * C, C)],
                                 g[sl], sd[sl])

        def wait_data(sl):
            pltpu.make_async_copy(data_hbm.at[pl.ds(0, C)], g[sl],
                                  sd[sl]).wait()

        def scat(sl, ir):
            pltpu.async_copy(g[sl], acc.at[b[ir].at[0]], ts[sl], add=True)

        def wait_scat(sl):
            pltpu.make_async_copy(data_hbm.at[pl.ds(0, C)], g[sl],
                                  ts[sl]).wait()

        for r in range(4):
            stage_idx(r, r)
        wait_idx(0)
        stage_data(0, 0, 0)
        wait_idx(1)
        stage_data(1, 1, 1)

        def proc(t, sl, ir, nxt_idx, nxt_data):
            wait_data(sl)
            scat(sl, ir)
            wait_scat(sl)
            if nxt_idx:
                stage_idx(t + 4, ir)
            if nxt_data:
                wait_idx((ir + 2) % 4)
                stage_data(t + 2, sl, (ir + 2) % 4)

        def body(k4, carry):
            t0 = 4 * k4
            for u in range(4):
                proc(t0 + u, u % 2, u, True, True)
            return carry
        K = (CHW - 4) // 4
        lax.fori_loop(0, K, body, 0)

        for t in range(4 * K, CHW):
            proc(t, t % 2, t % 4, t + 4 < CHW, t + 2 < CHW)
        plsc.subcore_barrier()

        for q in range(stripe // C):
            base = sid * stripe + q * C
            pltpu.sync_copy(acc.at[pl.ds(base, C)],
                            part_hbm.at[pl.ds(cid * NPAD + base, C)])

    return k(*in_arrays)


def _node_mlp(out_nodes, pg, pm, NPAD, W1a, W1b, b1, W2, b2):
    N, D = out_nodes.shape
    BN = 1024
    assert NPAD % BN == 0
    nb = NPAD // BN
    grid = (N + BN - 1) // BN

    def body(o_ref, a_ref, b_ref, c_ref, d_ref, w1a_ref, w1b_ref, b1_ref,
             w2_ref, b2_ref, y_ref):
        x = o_ref[...]
        aggr = a_ref[...] + b_ref[...] + c_ref[...] + d_ref[...]
        h = jnp.maximum(
            jnp.dot(x, w1a_ref[...], preferred_element_type=jnp.float32)
            + jnp.dot(aggr, w1b_ref[...], preferred_element_type=jnp.float32)
            + b1_ref[...], 0.0)
        y_ref[...] = (
            x + jnp.dot(h, w2_ref[...], preferred_element_type=jnp.float32)
            + b2_ref[...]
        )

    return pl.pallas_call(
        body,
        grid=(grid,),
        in_specs=[
            pl.BlockSpec((BN, D), lambda i: (i, 0)),
            pl.BlockSpec((BN, D), lambda i: (i, 0)),
            pl.BlockSpec((BN, D), lambda i: (nb + i, 0)),
            pl.BlockSpec((BN, D), lambda i: (i, 0)),
            pl.BlockSpec((BN, D), lambda i: (nb + i, 0)),
            pl.BlockSpec((D, D), lambda i: (0, 0)),
            pl.BlockSpec((D, D), lambda i: (0, 0)),
            pl.BlockSpec((1, D), lambda i: (0, 0)),
            pl.BlockSpec((D, D), lambda i: (0, 0)),
            pl.BlockSpec((1, D), lambda i: (0, 0)),
        ],
        out_specs=pl.BlockSpec((BN, D), lambda i: (i, 0)),
        out_shape=jax.ShapeDtypeStruct((N, D), jnp.float32),
    )(out_nodes, pg, pg, pm, pm, W1a, W1b, b1.reshape(1, D), W2,
      b2.reshape(1, D))


def kernel(z, edge_index, edge_attr,
           W1_0, b1_0, W2_0, b2_0, We1_0, be1_0, We2_0, be2_0,
           W1_1, b1_1, W2_1, b2_1, We1_1, be1_1, We2_1, be2_1):
    N, D = z.shape
    E = edge_index.shape[1]
    NW = _NC * _NS
    assert E % (NW * _C) == 0
    NCH = E // _C
    iiv = edge_index[0].reshape(NCH, 1, _C)
    jjv = edge_index[1].reshape(NCH, 1, _C)

    params = [
        (W1_0, b1_0, W2_0, b2_0, We1_0, be1_0, We2_0, be2_0),
        (W1_1, b1_1, W2_1, b2_1, We1_1, be1_1, We2_1, be2_1),
    ]
    msgs = [_edge_mlp(edge_attr, p[4], p[5], p[6], p[7]) for p in params]

    NPAD = _npad(N)
    out = z
    for l, (W1, b1, W2, b2, _, _, _, _) in enumerate(params):
        pg = _sc_pass(out, iiv, jjv, NPAD)
        pm = _sc_pass(msgs[l], iiv, None, NPAD)
        out = _node_mlp(out, pg, pm, NPAD, W1[:D], W1[D:], b1, W2, b2)
    return out


# speedup vs baseline: 1.0430x; 1.0430x over previous
import functools
import jax
import jax.numpy as jnp
from jax import lax
from jax.experimental import pallas as pl
from jax.experimental.pallas import tpu as pltpu
from jax.experimental.pallas import tpu_sc as plsc

_NC = 2
_NS = 16
_C = 80


def _edge_mlp(ea, We1, be1, We2, be2):
    E, F = ea.shape
    D = We1.shape[1]
    BE = 2000
    assert E % BE == 0

    def body(ea_ref, w1_ref, b1_ref, w2_ref, b2_ref, o_ref):
        ea_b = ea_ref[...]
        u = jnp.broadcast_to(b1_ref[...], (BE, D))
        for k in range(F):
            u = u + ea_b[:, k:k + 1] * w1_ref[k:k + 1, :]
        h = jnp.maximum(u, 0.0)
        o_ref[...] = (
            jnp.dot(h, w2_ref[...], preferred_element_type=jnp.float32)
            + b2_ref[...]
        )

    return pl.pallas_call(
        body,
        grid=(E // BE,),
        in_specs=[
            pl.BlockSpec((BE, F), lambda i: (i, 0)),
            pl.BlockSpec((F, D), lambda i: (0, 0)),
            pl.BlockSpec((1, D), lambda i: (0, 0)),
            pl.BlockSpec((D, D), lambda i: (0, 0)),
            pl.BlockSpec((1, D), lambda i: (0, 0)),
        ],
        out_specs=pl.BlockSpec((BE, D), lambda i: (i, 0)),
        out_shape=jax.ShapeDtypeStruct((E, D), jnp.float32),
    )(ea, We1, be1.reshape(1, D), We2, be2.reshape(1, D))


def _sc_message_pass(out_nodes, m_edges, iiv, jjv):
    N, D = out_nodes.shape
    NCH, one, C = iiv.shape
    NW = _NC * _NS
    CHW = NCH // NW
    EW = CHW * C
    NPAD = ((N + 2048 - 1) // 2048) * 2048
    stripe = NPAD // _NS
    assert stripe % C == 0
    mesh = plsc.VectorSubcoreMesh(core_axis_name="c", subcore_axis_name="s")

    @functools.partial(
        pl.kernel,
        out_type=jax.ShapeDtypeStruct((_NC * NPAD, D), jnp.float32),
        mesh=mesh,
        scratch_types=[
            pltpu.VMEM((2, C), jnp.int32),
            pltpu.VMEM((2, C), jnp.int32),
            pltpu.VMEM((2, C), jnp.int32),
            pltpu.VMEM((2, C), jnp.int32),
            pltpu.VMEM((C, D), jnp.float32),
            pltpu.VMEM((C, D), jnp.float32),
            pltpu.VMEM((C, D), jnp.float32),
            pltpu.VMEM((C, D), jnp.float32),
            pltpu.VMEM_SHARED((NPAD, D), jnp.float32),
        ] + [pltpu.SemaphoreType.DMA] * 12,
    )
    def k(out_hbm, m_hbm, ii_hbm, jj_hbm, part_hbm,
          b0, b1, b2, b3, g0, g1, m0, m1, acc,
          si0, si1, si2, si3, sg0, sg1, sm0, sm1, tg0, tg1, tm0, tm1):
        cid = lax.axis_index("c")
        sid = lax.axis_index("s")
        wid = sid * _NC + cid
        b = [b0, b1, b2, b3]
        g = [g0, g1]
        m = [m0, m1]
        si = [si0, si1, si2, si3]
        sg = [sg0, sg1]
        sm = [sm0, sm1]
        tg = [tg0, tg1]
        tm = [tm0, tm1]

        def zrow(r, carry):
            for kk in range(D // 16):
                g0[r, pl.ds(kk * 16, 16)] = jnp.zeros((16,), jnp.float32)
            return carry
        lax.fori_loop(0, C, zrow, 0)
        for q in range(stripe // C):
            pltpu.sync_copy(g0, acc.at[pl.ds(sid * stripe + q * C, C)])
        plsc.subcore_barrier()

        def stage_idx(t, ir):
            pltpu.async_copy(ii_hbm.at[wid * CHW + t],
                             b[ir].at[pl.ds(0, 1)], si[ir])
            pltpu.async_copy(jj_hbm.at[wid * CHW + t],
                             b[ir].at[pl.ds(1, 1)], si[ir])

        def wait_idx(ir):
            for _ in range(2):
                pltpu.make_async_copy(ii_hbm.at[0], b[ir].at[pl.ds(0, 1)],
                                      si[ir]).wait()

        def stage_data(t, sl, ir):
            pltpu.async_copy(out_hbm.at[b[ir].at[1]], g[sl], sg[sl])
            pltpu.async_copy(m_hbm.at[pl.ds(wid * EW + t * C, C)],
                             m[sl], sm[sl])

        def wait_data(sl):
            pltpu.make_async_copy(m_hbm.at[pl.ds(0, C)], g[sl],
                                  sg[sl]).wait()
            pltpu.make_async_copy(m_hbm.at[pl.ds(0, C)], m[sl],
                                  sm[sl]).wait()

        def scat(sl, ir):
            pltpu.async_copy(g[sl], acc.at[b[ir].at[0]], tg[sl], add=True)
            pltpu.async_copy(m[sl], acc.at[b[ir].at[0]], tm[sl], add=True)

        def wait_scat(sl):
            pltpu.make_async_copy(m_hbm.at[pl.ds(0, C)], g[sl],
                                  tg[sl]).wait()
            pltpu.make_async_copy(m_hbm.at[pl.ds(0, C)], m[sl],
                                  tm[sl]).wait()

        for r in range(4):
            stage_idx(r, r)
        wait_idx(0)
        stage_data(0, 0, 0)
        wait_idx(1)
        stage_data(1, 1, 1)

        def proc(t, sl, ir, nxt_idx, nxt_data):
            wait_data(sl)
            scat(sl, ir)
            wait_scat(sl)
            if nxt_idx:
                stage_idx(t + 4, ir)
            if nxt_data:
                wait_idx((ir + 2) % 4)
                stage_data(t + 2, sl, (ir + 2) % 4)

        def body(k4, carry):
            t0 = 4 * k4
            for u in range(4):
                proc(t0 + u, u % 2, u, True, True)
            return carry
        K = (CHW - 4) // 4
        lax.fori_loop(0, K, body, 0)

        for t in range(4 * K, CHW):
            proc(t, t % 2, t % 4, t + 4 < CHW, t + 2 < CHW)
        plsc.subcore_barrier()

        for q in range(stripe // C):
            base = sid * stripe + q * C
            pltpu.sync_copy(acc.at[pl.ds(base, C)],
                            part_hbm.at[pl.ds(cid * NPAD + base, C)])

    return k(out_nodes, m_edges, iiv, jjv)


def _node_mlp(out_nodes, part, NPAD, W1a, W1b, b1, W2, b2):
    N, D = out_nodes.shape
    BN = 1024
    assert NPAD % BN == 0
    nb = NPAD // BN
    grid = (N + BN - 1) // BN

    def body(o_ref, p0_ref, p1_ref, w1a_ref, w1b_ref, b1_ref, w2_ref,
             b2_ref, y_ref):
        x = o_ref[...]
        aggr = p0_ref[...] + p1_ref[...]
        h = jnp.maximum(
            jnp.dot(x, w1a_ref[...], preferred_element_type=jnp.float32)
            + jnp.dot(aggr, w1b_ref[...], preferred_element_type=jnp.float32)
            + b1_ref[...], 0.0)
        y_ref[...] = (
            x + jnp.dot(h, w2_ref[...], preferred_element_type=jnp.float32)
            + b2_ref[...]
        )

    return pl.pallas_call(
        body,
        grid=(grid,),
        in_specs=[
            pl.BlockSpec((BN, D), lambda i: (i, 0)),
            pl.BlockSpec((BN, D), lambda i: (i, 0)),
            pl.BlockSpec((BN, D), lambda i: (nb + i, 0)),
            pl.BlockSpec((D, D), lambda i: (0, 0)),
            pl.BlockSpec((D, D), lambda i: (0, 0)),
            pl.BlockSpec((1, D), lambda i: (0, 0)),
            pl.BlockSpec((D, D), lambda i: (0, 0)),
            pl.BlockSpec((1, D), lambda i: (0, 0)),
        ],
        out_specs=pl.BlockSpec((BN, D), lambda i: (i, 0)),
        out_shape=jax.ShapeDtypeStruct((N, D), jnp.float32),
    )(out_nodes, part, part, W1a, W1b, b1.reshape(1, D), W2,
      b2.reshape(1, D))


def kernel(z, edge_index, edge_attr,
           W1_0, b1_0, W2_0, b2_0, We1_0, be1_0, We2_0, be2_0,
           W1_1, b1_1, W2_1, b2_1, We1_1, be1_1, We2_1, be2_1):
    N, D = z.shape
    E = edge_index.shape[1]
    NW = _NC * _NS
    assert E % (NW * _C) == 0
    NCH = E // _C
    iiv = edge_index[0].reshape(NCH, 1, _C)
    jjv = edge_index[1].reshape(NCH, 1, _C)

    params = [
        (W1_0, b1_0, W2_0, b2_0, We1_0, be1_0, We2_0, be2_0),
        (W1_1, b1_1, W2_1, b2_1, We1_1, be1_1, We2_1, be2_1),
    ]
    msgs = [_edge_mlp(edge_attr, p[4], p[5], p[6], p[7]) for p in params]

    NPAD = ((N + 2048 - 1) // 2048) * 2048
    out = z
    for l, (W1, b1, W2, b2, _, _, _, _) in enumerate(params):
        part = _sc_message_pass(out, msgs[l], iiv, jjv)
        out = _node_mlp(out, part, NPAD, W1[:D], W1[D:], b1, W2, b2)
    return out


# speedup vs baseline: 1.2117x; 1.1617x over previous
import functools
import jax
import jax.numpy as jnp
from jax import lax
from jax.experimental import pallas as pl
from jax.experimental.pallas import tpu as pltpu
from jax.experimental.pallas import tpu_sc as plsc

_NC = 2
_NS = 16
_C = 80


def _edge_mlp(ea, We1, be1, We2, be2):
    E, F = ea.shape
    D = We1.shape[1]
    BE = 2000
    assert E % BE == 0

    def body(ea_ref, w1_ref, b1_ref, w2_ref, b2_ref, o_ref):
        u = (jnp.dot(ea_ref[...], w1_ref[...],
                     preferred_element_type=jnp.float32) + b1_ref[...])
        h = jnp.maximum(u, 0.0)
        o_ref[...] = (
            jnp.dot(h, w2_ref[...], preferred_element_type=jnp.float32)
            + b2_ref[...]
        )

    return pl.pallas_call(
        body,
        grid=(E // BE,),
        in_specs=[
            pl.BlockSpec((BE, F), lambda i: (i, 0)),
            pl.BlockSpec((F, D), lambda i: (0, 0)),
            pl.BlockSpec((1, D), lambda i: (0, 0)),
            pl.BlockSpec((D, D), lambda i: (0, 0)),
            pl.BlockSpec((1, D), lambda i: (0, 0)),
        ],
        out_specs=pl.BlockSpec((BE, D), lambda i: (i, 0)),
        out_shape=jax.ShapeDtypeStruct((E, D), jnp.float32),
    )(ea, We1, be1.reshape(1, D), We2, be2.reshape(1, D))


def _sc_message_pass(out_nodes, m_edges, iiv, jjv):
    N, D = out_nodes.shape
    C = _C
    NCH = iiv.shape[0] // C
    NW = _NC * _NS
    CHW = NCH // NW
    EW = CHW * C
    NPAD = ((N + 2048 - 1) // 2048) * 2048
    stripe = NPAD // _NS
    assert stripe % C == 0
    mesh = plsc.VectorSubcoreMesh(core_axis_name="c", subcore_axis_name="s")

    @functools.partial(
        pl.kernel,
        out_type=jax.ShapeDtypeStruct((_NC * NPAD, D), jnp.float32),
        mesh=mesh,
        scratch_types=[
            pltpu.VMEM((C,), jnp.int32),
            pltpu.VMEM((C,), jnp.int32),
            pltpu.VMEM((C,), jnp.int32),
            pltpu.VMEM((C,), jnp.int32),
            pltpu.VMEM((C,), jnp.int32),
            pltpu.VMEM((C,), jnp.int32),
            pltpu.VMEM((C,), jnp.int32),
            pltpu.VMEM((C,), jnp.int32),
            pltpu.VMEM((C, D), jnp.float32),
            pltpu.VMEM((C, D), jnp.float32),
            pltpu.VMEM((C, D), jnp.float32),
            pltpu.VMEM((C, D), jnp.float32),
            pltpu.VMEM_SHARED((NPAD, D), jnp.float32),
        ] + [pltpu.SemaphoreType.DMA] * 12,
    )
    def k(out_hbm, m_hbm, ii_hbm, jj_hbm, part_hbm,
          bi0, bi1, bi2, bi3, bj0, bj1, bj2, bj3, g0, g1, m0, m1, acc,
          si0, si1, si2, si3, sg0, sg1, sm0, sm1, tg0, tg1, tm0, tm1):
        cid = lax.axis_index("c")
        sid = lax.axis_index("s")
        wid = sid * _NC + cid
        bi = [bi0, bi1, bi2, bi3]
        bj = [bj0, bj1, bj2, bj3]
        g = [g0, g1]
        m = [m0, m1]
        si = [si0, si1, si2, si3]
        sg = [sg0, sg1]
        sm = [sm0, sm1]
        tg = [tg0, tg1]
        tm = [tm0, tm1]

        def zrow(r, carry):
            for kk in range(D // 16):
                g0[r, pl.ds(kk * 16, 16)] = jnp.zeros((16,), jnp.float32)
            return carry
        lax.fori_loop(0, C, zrow, 0)
        for q in range(stripe // C):
            pltpu.sync_copy(g0, acc.at[pl.ds(sid * stripe + q * C, C)])
        plsc.subcore_barrier()

        def stage_idx(t, ir):
            pltpu.async_copy(ii_hbm.at[pl.ds(wid * EW + t * C, C)],
                             bi[ir], si[ir])
            pltpu.async_copy(jj_hbm.at[pl.ds(wid * EW + t * C, C)],
                             bj[ir], si[ir])

        def wait_idx(ir):
            for _ in range(2):
                pltpu.make_async_copy(ii_hbm.at[pl.ds(0, C)], bi[ir],
                                      si[ir]).wait()

        def stage_data(t, sl, ir):
            pltpu.async_copy(out_hbm.at[bj[ir]], g[sl], sg[sl])
            pltpu.async_copy(m_hbm.at[pl.ds(wid * EW + t * C, C)],
                             m[sl], sm[sl])

        def wait_data(sl):
            pltpu.make_async_copy(m_hbm.at[pl.ds(0, C)], g[sl],
                                  sg[sl]).wait()
            pltpu.make_async_copy(m_hbm.at[pl.ds(0, C)], m[sl],
                                  sm[sl]).wait()

        def scat(sl, ir):
            pltpu.async_copy(g[sl], acc.at[bi[ir]], tg[sl], add=True)
            pltpu.async_copy(m[sl], acc.at[bi[ir]], tm[sl], add=True)

        def wait_scat(sl):
            pltpu.make_async_copy(m_hbm.at[pl.ds(0, C)], g[sl],
                                  tg[sl]).wait()
            pltpu.make_async_copy(m_hbm.at[pl.ds(0, C)], m[sl],
                                  tm[sl]).wait()

        for r in range(4):
            stage_idx(r, r)
        wait_idx(0)
        stage_data(0, 0, 0)
        wait_idx(1)
        stage_data(1, 1, 1)

        def proc(t, sl, ir, nxt_idx, nxt_data):
            wait_data(sl)
            scat(sl, ir)
            wait_scat(sl)
            if nxt_idx:
                stage_idx(t + 4, ir)
            if nxt_data:
                wait_idx((ir + 2) % 4)
                stage_data(t + 2, sl, (ir + 2) % 4)

        def body(k4, carry):
            t0 = 4 * k4
            for u in range(4):
                proc(t0 + u, u % 2, u, True, True)
            return carry
        K = (CHW - 4) // 4
        lax.fori_loop(0, K, body, 0)

        for t in range(4 * K, CHW):
            proc(t, t % 2, t % 4, t + 4 < CHW, t + 2 < CHW)
        plsc.subcore_barrier()

        for q in range(stripe // C):
            base = sid * stripe + q * C
            pltpu.sync_copy(acc.at[pl.ds(base, C)],
                            part_hbm.at[pl.ds(cid * NPAD + base, C)])

    return k(out_nodes, m_edges, iiv, jjv)


def _node_mlp(out_nodes, part, NPAD, W1a, W1b, b1, W2, b2):
    N, D = out_nodes.shape
    BN = 1024
    assert NPAD % BN == 0
    nb = NPAD // BN
    grid = (N + BN - 1) // BN

    def body(o_ref, p0_ref, p1_ref, w1a_ref, w1b_ref, b1_ref, w2_ref,
             b2_ref, y_ref):
        x = o_ref[...]
        aggr = p0_ref[...] + p1_ref[...]
        h = jnp.maximum(
            jnp.dot(x, w1a_ref[...], preferred_element_type=jnp.float32)
            + jnp.dot(aggr, w1b_ref[...], preferred_element_type=jnp.float32)
            + b1_ref[...], 0.0)
        y_ref[...] = (
            x + jnp.dot(h, w2_ref[...], preferred_element_type=jnp.float32)
            + b2_ref[...]
        )

    return pl.pallas_call(
        body,
        grid=(grid,),
        in_specs=[
            pl.BlockSpec((BN, D), lambda i: (i, 0)),
            pl.BlockSpec((BN, D), lambda i: (i, 0)),
            pl.BlockSpec((BN, D), lambda i: (nb + i, 0)),
            pl.BlockSpec((D, D), lambda i: (0, 0)),
            pl.BlockSpec((D, D), lambda i: (0, 0)),
            pl.BlockSpec((1, D), lambda i: (0, 0)),
            pl.BlockSpec((D, D), lambda i: (0, 0)),
            pl.BlockSpec((1, D), lambda i: (0, 0)),
        ],
        out_specs=pl.BlockSpec((BN, D), lambda i: (i, 0)),
        out_shape=jax.ShapeDtypeStruct((N, D), jnp.float32),
    )(out_nodes, part, part, W1a, W1b, b1.reshape(1, D), W2,
      b2.reshape(1, D))


def kernel(z, edge_index, edge_attr,
           W1_0, b1_0, W2_0, b2_0, We1_0, be1_0, We2_0, be2_0,
           W1_1, b1_1, W2_1, b2_1, We1_1, be1_1, We2_1, be2_1):
    N, D = z.shape
    E = edge_index.shape[1]
    NW = _NC * _NS
    assert E % (NW * _C) == 0
    iiv = edge_index[0]
    jjv = edge_index[1]

    params = [
        (W1_0, b1_0, W2_0, b2_0, We1_0, be1_0, We2_0, be2_0),
        (W1_1, b1_1, W2_1, b2_1, We1_1, be1_1, We2_1, be2_1),
    ]
    msgs = [_edge_mlp(edge_attr, p[4], p[5], p[6], p[7]) for p in params]

    NPAD = ((N + 2048 - 1) // 2048) * 2048
    out = z
    for l, (W1, b1, W2, b2, _, _, _, _) in enumerate(params):
        part = _sc_message_pass(out, msgs[l], iiv, jjv)
        out = _node_mlp(out, part, NPAD, W1[:D], W1[D:], b1, W2, b2)
    return out
